# Initial kernel scaffold; baseline (speedup 1.0000x reference)
import jax
import jax.numpy as jnp
from jax.experimental import pallas as pl


def kernel(x, edge_index, batch_index, params):
    raise NotImplementedError("write your pallas kernel here")


# speedup vs baseline: 2.0560x; 2.0560x over previous
import functools

import jax
import jax.numpy as jnp
import numpy as np
from jax import lax
from jax.experimental import pallas as pl
from jax.experimental.pallas import tpu as pltpu
from jax.experimental.pallas import tpu_sc as plsc

N = 10000
NP = 10240
E = 320000
EP = 327680
G = 16
DUMMY = N
NC, NS = 2, 16
NW = NC * NS
EPW = EP // NW
CH = 128
NCH = EPW // CH
SUBR = NP // NS
BLK = 512
NEB = EP // BLK
NNB = NP // BLK
F32 = jnp.float32
I32 = jnp.int32


def _mesh():
    return plsc.VectorSubcoreMesh(core_axis_name="c", subcore_axis_name="s")


def _sc_gather2(table, send, recv):
    @functools.partial(
        pl.kernel,
        out_type=[
            jax.ShapeDtypeStruct((EP, 128), F32),
            jax.ShapeDtypeStruct((EP, 128), F32),
        ],
        mesh=_mesh(),
        scratch_types=[
            pltpu.VMEM((CH,), I32),
            pltpu.VMEM((CH,), I32),
            pltpu.VMEM((CH, 128), F32),
            pltpu.VMEM((CH, 128), F32),
            pltpu.SemaphoreType.DMA,
        ],
    )
    def k(tab, sidx, ridx, xs_o, xr_o, si_v, ri_v, sr_v, rr_v, sem):
        wid = lax.axis_index("s") * NC + lax.axis_index("c")
        base = wid * EPW

        def body(c, carry):
            off = pl.multiple_of(base + c * CH, CH)
            pltpu.sync_copy(sidx.at[pl.ds(off, CH)], si_v)
            pltpu.sync_copy(ridx.at[pl.ds(off, CH)], ri_v)
            pltpu.async_copy(tab.at[si_v], sr_v, sem).wait()
            pltpu.async_copy(tab.at[ri_v], rr_v, sem).wait()
            pltpu.sync_copy(sr_v, xs_o.at[pl.ds(off, CH)])
            pltpu.sync_copy(rr_v, xr_o.at[pl.ds(off, CH)])
            return carry

        lax.fori_loop(0, NCH, body, 0)

    return k(table, send, recv)


def _sc_scatter(msgs, midx):
    zeros = jnp.zeros((NP, 128), F32)

    @functools.partial(
        pl.kernel,
        out_type=jax.ShapeDtypeStruct((NC, NP, 128), F32),
        mesh=_mesh(),
        scratch_types=[
            pltpu.VMEM((CH,), I32),
            pltpu.VMEM((CH, 128), F32),
            pltpu.VMEM_SHARED((NP, 128), F32),
            pltpu.SemaphoreType.DMA,
        ],
    )
    def k(m_h, mi_h, z_h, out_h, idx_v, rows_v, acc, sem):
        cid = lax.axis_index("c")
        sid = lax.axis_index("s")
        wid = sid * NC + cid
        base = wid * EPW
        pltpu.sync_copy(z_h.at[pl.ds(sid * SUBR, SUBR)],
                        acc.at[pl.ds(sid * SUBR, SUBR)])
        plsc.subcore_barrier()

        def body(c, carry):
            off = pl.multiple_of(base + c * CH, CH)
            pltpu.sync_copy(mi_h.at[pl.ds(off, CH)], idx_v)
            pltpu.sync_copy(m_h.at[pl.ds(off, CH)], rows_v)
            pltpu.sync_copy(rows_v, acc.at[idx_v], add=True)
            return carry

        lax.fori_loop(0, NCH, body, 0)
        plsc.subcore_barrier()
        pltpu.sync_copy(acc.at[pl.ds(sid * SUBR, SUBR)],
                        out_h.at[cid, pl.ds(sid * SUBR, SUBR)])

    return k(msgs, midx, zeros)


def _sc_sage(tables, send, midx):
    nh = len(tables)
    zeros = jnp.zeros((NP, 128), F32)

    @functools.partial(
        pl.kernel,
        out_type=jax.ShapeDtypeStruct((NC, nh, NP, 128), F32),
        mesh=_mesh(),
        scratch_types=[
            pltpu.VMEM((CH,), I32),
            pltpu.VMEM((CH,), I32),
            pltpu.VMEM((CH, 128), F32),
            pltpu.VMEM_SHARED((NP, 128), F32),
            pltpu.SemaphoreType.DMA,
        ],
    )
    def k(*args):
        tabs = args[:nh]
        sidx, mi_h, z_h, out_h, gi_v, si_v, rows_v, acc, sem = args[nh:]
        cid = lax.axis_index("c")
        sid = lax.axis_index("s")
        wid = sid * NC + cid
        base = wid * EPW
        for h in range(nh):
            pltpu.sync_copy(z_h.at[pl.ds(sid * SUBR, SUBR)],
                            acc.at[pl.ds(sid * SUBR, SUBR)])
            plsc.subcore_barrier()

            def body(c, carry):
                off = pl.multiple_of(base + c * CH, CH)
                pltpu.sync_copy(sidx.at[pl.ds(off, CH)], gi_v)
                pltpu.sync_copy(mi_h.at[pl.ds(off, CH)], si_v)
                pltpu.async_copy(tabs[h].at[gi_v], rows_v, sem).wait()
                pltpu.sync_copy(rows_v, acc.at[si_v], add=True)
                return carry

            lax.fori_loop(0, NCH, body, 0)
            plsc.subcore_barrier()
            pltpu.sync_copy(acc.at[pl.ds(sid * SUBR, SUBR)],
                            out_h.at[cid, h, pl.ds(sid * SUBR, SUBR)])
            plsc.subcore_barrier()

    return k(*tables, send, midx, zeros)


def _full(shape):
    return pl.BlockSpec(shape, lambda i: tuple(0 for _ in shape))


def _rows(shape):
    return pl.BlockSpec(shape, lambda i: (i,) + tuple(0 for _ in shape[1:]))


def _relu(v):
    return jnp.maximum(v, 0.0)


def _dot(a, b):
    return jnp.dot(a, b, preferred_element_type=F32)


def _tc_edge0(xr, xs, recv2d, wr, ws, we, b1, w2, b2):
    def body(xr_r, xs_r, rv_r, wr_r, ws_r, we_r, b1_r, w2_r, b2_r,
             e_r, m_r, mi_r):
        a = xr_r[...]
        b = xs_r[...]
        d = a - b
        d0 = d[:, 0:1]
        d1 = d[:, 1:2]
        d2 = d[:, 2:3]
        dist = jnp.sqrt(d0 * d0 + d1 * d1 + d2 * d2)
        inv = jnp.where(dist > 0.0, 1.0 / jnp.where(dist == 0.0, 1.0, dist),
                        0.0)
        e_r[:, 0:3] = d[:, 3:6]
        e_r[:, 3:4] = dist
        e_r[:, 4:7] = d[:, 0:3] * inv
        e_r[:, 7:8] = jnp.zeros_like(dist)
        e = e_r[...]
        h = _relu(_dot(a, wr_r[...]) + _dot(b, ws_r[...]) + _dot(e, we_r[...])
                  + b1_r[...])
        m_r[...] = _relu(_dot(h, w2_r[...]) + b2_r[...])
        mi_r[...] = jnp.where(b[:, 3:4] <= a[:, 3:4], rv_r[...], DUMMY)

    return pl.pallas_call(
        body,
        grid=(NEB,),
        in_specs=[_rows((BLK, 128)), _rows((BLK, 128)), _rows((BLK, 1)),
                  _full((128, 256)), _full((128, 256)), _full((8, 256)),
                  _full((1, 256)), _full((256, 128)), _full((1, 128))],
        out_specs=[_rows((BLK, 8)), _rows((BLK, 128)), _rows((BLK, 1))],
        out_shape=[jax.ShapeDtypeStruct((EP, 8), F32),
                   jax.ShapeDtypeStruct((EP, 128), F32),
                   jax.ShapeDtypeStruct((EP, 1), I32)],
    )(xr, xs, recv2d, wr, ws, we, b1, w2, b2)


def _tc_edge_mlp(xr, xs, e8, wr, ws, we, b1, w2, b2):
    def body(xr_r, xs_r, e_r, wr_r, ws_r, we_r, b1_r, w2_r, b2_r, m_r):
        h = _relu(_dot(xr_r[...], wr_r[...]) + _dot(xs_r[...], ws_r[...])
                  + _dot(e_r[...], we_r[...]) + b1_r[...])
        m_r[...] = _relu(_dot(h, w2_r[...]) + b2_r[...])

    return pl.pallas_call(
        body,
        grid=(NEB,),
        in_specs=[_rows((BLK, 128)), _rows((BLK, 128)), _rows((BLK, 8)),
                  _full((128, 256)), _full((128, 256)), _full((8, 256)),
                  _full((1, 256)), _full((256, 128)), _full((1, 128))],
        out_specs=_rows((BLK, 128)),
        out_shape=jax.ShapeDtypeStruct((EP, 128), F32),
    )(xr, xs, e8, wr, ws, we, b1, w2, b2)


def _tc_upd(a, w1, b1, w2, b2, cnt_flag):
    def body(a0_r, a1_r, w1_r, b1_r, w2_r, b2_r, o_r):
        s = a0_r[0] + a1_r[0]
        h = _relu(_dot(s, w1_r[...]) + b1_r[...])
        h = _relu(_dot(h, w2_r[...]) + b2_r[...])
        o_r[:, 0:64] = h
        cc = lax.broadcasted_iota(I32, (BLK, 64), 1)
        o_r[:, 64:128] = jnp.where(cc == 0, cnt_flag, 0.0)

    return pl.pallas_call(
        body,
        grid=(NNB,),
        in_specs=[pl.BlockSpec((1, BLK, 128), lambda i: (0, i, 0)),
                  pl.BlockSpec((1, BLK, 128), lambda i: (1, i, 0)),
                  _full((128, 128)), _full((1, 128)),
                  _full((128, 64)), _full((1, 64))],
        out_specs=_rows((BLK, 128)),
        out_shape=jax.ShapeDtypeStruct((NP, 128), F32),
    )(a, a, w1, b1, w2, b2)


def _tc_add(a):
    def body(a0_r, a1_r, o_r):
        o_r[...] = a0_r[0] + a1_r[0]

    return pl.pallas_call(
        body,
        grid=(NNB,),
        in_specs=[pl.BlockSpec((1, BLK, 128), lambda i: (0, i, 0)),
                  pl.BlockSpec((1, BLK, 128), lambda i: (1, i, 0))],
        out_specs=_rows((BLK, 128)),
        out_shape=jax.ShapeDtypeStruct((NP, 128), F32),
    )(a, a)


def _tc_sage1(h2t, s1, wx, wa, b):
    def body(h_r, s0_r, s1_r, wx_r, wa_r, b_r, o1_r, o2_r, c_r):
        s = s0_r[0, 0] + s1_r[0, 0]
        cnt = s[:, 64:72]
        c = jnp.maximum(cnt[:, 0:1], 1.0)
        agg = s[:, 0:64] / c
        z = (b_r[...] + _dot(h_r[:, 0:64], wx_r[...]) + _dot(agg, wa_r[...]))
        z = z * lax.rsqrt(jnp.maximum(jnp.sum(z * z, axis=1, keepdims=True),
                                      1e-12))
        z = _relu(z)
        o1_r[...] = z[:, 0:128]
        o2_r[...] = z[:, 128:256]
        c_r[...] = cnt

    return pl.pallas_call(
        body,
        grid=(NNB,),
        in_specs=[_rows((BLK, 128)),
                  pl.BlockSpec((1, 1, BLK, 128), lambda i: (0, 0, i, 0)),
                  pl.BlockSpec((1, 1, BLK, 128), lambda i: (1, 0, i, 0)),
                  _full((64, 256)), _full((64, 256)), _full((1, 256))],
        out_specs=[_rows((BLK, 128)), _rows((BLK, 128)), _rows((BLK, 8))],
        out_shape=[jax.ShapeDtypeStruct((NP, 128), F32),
                   jax.ShapeDtypeStruct((NP, 128), F32),
                   jax.ShapeDtypeStruct((NP, 8), F32)],
    )(h2t, s1, s1, wx, wa, b)


def _tc_sage2(h3a, h3b, s2, cnt8, wx, wa, b):
    def body(ha_r, hb_r, s00_r, s10_r, s01_r, s11_r, c_r, wx_r, wa_r, b_r,
             *outs):
        c = jnp.maximum(c_r[:, 0:1], 1.0)
        agg0 = (s00_r[0, 0] + s10_r[0, 0]) / c
        agg1 = (s01_r[0, 0] + s11_r[0, 0]) / c
        z = (b_r[...]
             + _dot(ha_r[...], wx_r[pl.ds(0, 128)])
             + _dot(hb_r[...], wx_r[pl.ds(128, 128)])
             + _dot(agg0, wa_r[pl.ds(0, 128)])
             + _dot(agg1, wa_r[pl.ds(128, 128)]))
        z = z * lax.rsqrt(jnp.maximum(jnp.sum(z * z, axis=1, keepdims=True),
                                      1e-12))
        z = _relu(z)
        for o in range(4):
            outs[o][...] = z[:, o * 128:(o + 1) * 128]

    def spec(cc, hh):
        return pl.BlockSpec((1, 1, BLK, 128), lambda i: (cc, hh, i, 0))

    return pl.pallas_call(
        body,
        grid=(NNB,),
        in_specs=[_rows((BLK, 128)), _rows((BLK, 128)),
                  spec(0, 0), spec(1, 0), spec(0, 1), spec(1, 1),
                  _rows((BLK, 8)),
                  _full((256, 512)), _full((256, 512)), _full((1, 512))],
        out_specs=[_rows((BLK, 128))] * 4,
        out_shape=[jax.ShapeDtypeStruct((NP, 128), F32)] * 4,
    )(h3a, h3b, s2, s2, s2, s2, cnt8, wx, wa, b)


def _tc_pool(h4a, h4b, h4c, h4d, xpad, bcol):
    def body(ha_r, hb_r, hc_r, hd_r, x_r, b_r,
             hmax_r, hsum_r, xs_r, xq_r, xmx_r, xmn_r, cg_r):
        i = pl.program_id(0)

        @pl.when(i == 0)
        def _init():
            hmax_r[...] = jnp.full((G, 512), -jnp.inf, F32)
            hsum_r[...] = jnp.zeros((G, 512), F32)
            xs_r[...] = jnp.zeros((G, 8), F32)
            xq_r[...] = jnp.zeros((G, 8), F32)
            xmx_r[...] = jnp.full((G, 8), -jnp.inf, F32)
            xmn_r[...] = jnp.full((G, 8), jnp.inf, F32)
            cg_r[...] = jnp.zeros((G, 8), F32)

        hv = jnp.concatenate([ha_r[...], hb_r[...], hc_r[...], hd_r[...]],
                             axis=1)
        xv = x_r[...]
        bc = b_r[:, 0:1]
        giota = lax.broadcasted_iota(I32, (1, G), 1).astype(F32)
        oht = (bc == giota).astype(F32)
        dn = (((0,), (0,)), ((), ()))
        hsum_r[...] += lax.dot_general(oht, hv, dn, preferred_element_type=F32)
        xs_r[...] += lax.dot_general(oht, xv, dn, preferred_element_type=F32)
        xq_r[...] += lax.dot_general(oht, xv * xv, dn,
                                     preferred_element_type=F32)
        cg_r[...] += lax.dot_general(oht, jnp.ones((BLK, 8), F32), dn,
                                     preferred_element_type=F32)
        for g in range(G):
            mk = bc == float(g)
            hm = jnp.max(jnp.where(mk, hv, -jnp.inf), axis=0, keepdims=True)
            hmax_r[pl.ds(g, 1), :] = jnp.maximum(hmax_r[pl.ds(g, 1), :], hm)
            xm = jnp.max(jnp.where(mk, xv, -jnp.inf), axis=0, keepdims=True)
            xmx_r[pl.ds(g, 1), :] = jnp.maximum(xmx_r[pl.ds(g, 1), :], xm)
            xn = jnp.min(jnp.where(mk, xv, jnp.inf), axis=0, keepdims=True)
            xmn_r[pl.ds(g, 1), :] = jnp.minimum(xmn_r[pl.ds(g, 1), :], xn)

    shapes = [jax.ShapeDtypeStruct((G, 512), F32),
              jax.ShapeDtypeStruct((G, 512), F32)] + \
             [jax.ShapeDtypeStruct((G, 8), F32)] * 5
    return pl.pallas_call(
        body,
        grid=(NNB,),
        in_specs=[_rows((BLK, 128))] * 4 + [_rows((BLK, 8)), _rows((BLK, 8))],
        out_specs=[_full((G, 512)), _full((G, 512))] + [_full((G, 8))] * 5,
        out_shape=shapes,
    )(h4a, h4b, h4c, h4d, xpad, bcol)


def _tc_decoder(pools, wts):
    hmax, hsum, xs_, xq, xmx, xmn, cg = pools

    def body(hmax_r, hsum_r, xs_r, xq_r, xmx_r, xmn_r, cg_r, *o):
        (wp1_r, wp2_r, wps_r, wav_r, wvr_r, wmx_r, wmn_r, bd1_r, s1_r, t1_r,
         wd2_r, bd2_r, s2_r, t2_r,
         l0_r, lb0_r, l1_r, lb1_r, l2_r, lb2_r,
         a0_r, ab0_r, a1_r, ab1_r, a2_r, ab2_r, sc_r, scb_r,
         q0_r, qb0_r, q1_r, qb1_r, q2_r, qb2_r,
         loge_r, za_r, sig_r) = o
        cnt = jnp.maximum(cg_r[...], 1.0)
        c1 = cnt[:, 0:1]
        avg = xs_r[...] / cnt
        var = jnp.abs(xq_r[...] / cnt - avg * avg)
        p2 = hsum_r[...] / c1
        g1 = (_dot(hmax_r[...], wp1_r[...]) + _dot(p2, wp2_r[...])
              + _dot(hsum_r[...], wps_r[...]) + _dot(avg, wav_r[...])
              + _dot(var, wvr_r[...]) + _dot(xmx_r[...], wmx_r[...])
              + _dot(xmn_r[...], wmn_r[...]) + bd1_r[...])
        g1 = jnp.where(g1 >= 0.0, g1, 0.15 * g1) * s1_r[...] + t1_r[...]
        g2 = _dot(g1, wd2_r[...]) + bd2_r[...]
        g2 = jnp.where(g2 >= 0.0, g2, 0.15 * g2) * s2_r[...] + t2_r[...]
        loge = _dot(_dot(_dot(g2, l0_r[...]) + lb0_r[...], l1_r[...])
                    + lb1_r[...], l2_r[...]) + lb2_r[...]
        ang = _dot(_dot(_dot(g2, a0_r[...]) + ab0_r[...], a1_r[...])
                   + ab1_r[...], a2_r[...]) + ab2_r[...]
        za = jax.nn.sigmoid(_dot(ang, sc_r[...]) + scb_r[...])
        sig = jnp.abs(_dot(_dot(_dot(g2, q0_r[...]) + qb0_r[...], q1_r[...])
                           + qb1_r[...], q2_r[...]) + qb2_r[...]) + 1e-05
        loge_r[...] = loge
        za_r[...] = za
        sig_r[...] = sig

    out_shape = [jax.ShapeDtypeStruct((G, 8), F32)] * 3
    return pl.pallas_call(
        body,
        out_shape=out_shape,
    )(hmax, hsum, xs_, xq, xmx, xmn, cg, *wts)


def _pad_rows(w, rows):
    return jnp.pad(w, ((0, rows - w.shape[0]), (0, 0)))


def _split_msg(mp, d, sg, sb, dpad):
    w1 = mp["l1"]["w"]
    b1 = mp["l1"]["b"]
    wr = _pad_rows(w1[0:d], dpad)
    ws = _pad_rows(w1[d:2 * d], dpad)
    we = w1[2 * d:2 * d + 7]
    b1f = b1 + sb @ we
    wef = _pad_rows(sg[:, None] * we, 8)
    return (wr, ws, wef, b1f[None, :], mp["l2"]["w"], mp["l2"]["b"][None, :])


def kernel(x, edge_index, batch_index, params):
    p = params
    xpad8 = jnp.pad(x, ((0, NP - N), (0, 2)))
    xpad128 = jnp.pad(x, ((0, NP - N), (0, 128 - 6)))
    send = jnp.concatenate([edge_index[0],
                            jnp.full((EP - E,), DUMMY, I32)])
    recv = jnp.concatenate([edge_index[1],
                            jnp.full((EP - E,), DUMMY, I32)])
    bcol = jnp.broadcast_to(
        jnp.concatenate([batch_index,
                         jnp.full((NP - N,), G, I32)]).astype(F32)[:, None],
        (NP, 8))

    sg = p["e_bn"]["gamma"] / np.sqrt(1.0 + 1e-3)
    sb = p["e_bn"]["beta"]

    xs8, xr8 = _sc_gather2(xpad128, send, recv)

    m1w = _split_msg(p["mp1"]["msg"][0], 6, sg, sb, 128)
    e8, m1, midx2d = _tc_edge0(xr8, xs8, recv[:, None], *m1w)
    midx = midx2d.reshape(EP)
    a1 = _sc_scatter(m1, midx)
    u1 = p["mp1"]["upd"]
    h1t = _tc_upd(a1, u1["l1"]["w"], u1["l1"]["b"][None, :],
                  u1["l2"]["w"], u1["l2"]["b"][None, :], 0.0)

    hs1, hr1 = _sc_gather2(h1t, send, recv)
    m2w = _split_msg(p["mp2"]["msg"][0], 64, sg, sb, 128)
    m2 = _tc_edge_mlp(hr1, hs1, e8, *m2w)
    a2 = _sc_scatter(m2, midx)
    x2 = _tc_add(a2)

    xs2, xr2 = _sc_gather2(x2, send, recv)
    m3w = _split_msg(p["mp2"]["msg"][1], 128, sg, sb, 128)
    m3 = _tc_edge_mlp(xr2, xs2, e8, *m3w)
    a3 = _sc_scatter(m3, midx)
    u2 = p["mp2"]["upd"]
    h2t = _tc_upd(a3, u2["l1"]["w"], u2["l1"]["b"][None, :],
                  u2["l2"]["w"], u2["l2"]["b"][None, :], 1.0)

    s1 = _sc_sage([h2t], send, midx)
    g1w = p["gcn1"]["w"]
    h3a, h3b, cnt8 = _tc_sage1(h2t, s1, g1w[0:64], g1w[64:128],
                               p["gcn1"]["b"][None, :])

    s2 = _sc_sage([h3a, h3b], send, midx)
    g2w = p["gcn2"]["w"]
    h4a, h4b, h4c, h4d = _tc_sage2(h3a, h3b, s2, cnt8,
                                   g2w[0:256], g2w[256:512],
                                   p["gcn2"]["b"][None, :])

    pools = _tc_pool(h4a, h4b, h4c, h4d, xpad8, bcol)

    d1w = p["dec1"]["w"]
    wavg = _pad_rows(d1w[1536:1542], 8)
    wvar = _pad_rows(d1w[1542:1548], 8)
    wmax = _pad_rows(d1w[1548:1554], 8)
    wmin = _pad_rows(d1w[1554:1560], 8)
    s1v = (p["bn1"]["gamma"] / np.sqrt(1.0 + 1e-3))[None, :]
    t1v = p["bn1"]["beta"][None, :]
    s2v = (p["bn2"]["gamma"] / np.sqrt(1.0 + 1e-3))[None, :]
    t2v = p["bn2"]["beta"][None, :]

    def head(pref):
        return [p[pref + "0"]["w"], p[pref + "0"]["b"][None, :],
                p[pref + "1"]["w"], p[pref + "1"]["b"][None, :],
                jnp.pad(p[pref + "_out"]["w"],
                        ((0, 0), (0, 8 - p[pref + "_out"]["w"].shape[1]))),
                jnp.pad(p[pref + "_out"]["b"],
                        (0, 8 - p[pref + "_out"]["b"].shape[0]))[None, :]]

    sc8 = jnp.pad(p["ang_scale"]["w"], ((0, 6), (0, 6)))
    scb8 = jnp.pad(p["ang_scale"]["b"], (0, 6))[None, :]
    wts = ([d1w[0:512], d1w[512:1024], d1w[1024:1536], wavg, wvar, wmax, wmin,
            p["dec1"]["b"][None, :], s1v, t1v,
            p["dec2"]["w"], p["dec2"]["b"][None, :], s2v, t2v]
           + head("loge") + head("ang") + [sc8, scb8] + head("sig"))
    loge8, za8, sig8 = _tc_decoder(pools, wts)

    xs_out = jnp.stack([loge8[:, 0], za8[:, 0] * np.pi,
                        za8[:, 1] * 2.0 * np.pi], axis=1)
    return jnp.concatenate([xs_out, sig8[:, 0:2]], axis=1)
